# baseline (speedup 1.0000x reference)
import functools

import jax
import jax.numpy as jnp
from jax import lax
from jax.experimental import pallas as pl
from jax.experimental.pallas import tpu as pltpu
from jax.experimental.pallas import tpu_sc as plsc

NUM_WORKERS = 32
LANES = 16


def _row_transform_body(x_ref, w_ref, b_ref, o_ref):
    o_ref[...] = (
        jnp.dot(x_ref[...], w_ref[...], preferred_element_type=jnp.float32)
        + b_ref[...]
    )


def _row_transform(table, W, b, block_rows):
    n, df = table.shape
    d = W.shape[1]
    grid = n // block_rows
    return pl.pallas_call(
        _row_transform_body,
        grid=(grid,),
        in_specs=[
            pl.BlockSpec((block_rows, df), lambda i: (i, 0)),
            pl.BlockSpec((df, d), lambda i: (0, 0)),
            pl.BlockSpec((1, d), lambda i: (0, 0)),
        ],
        out_specs=pl.BlockSpec((block_rows, d), lambda i: (i, 0)),
        out_shape=jax.ShapeDtypeStruct((n, d), jnp.float32),
    )(table, W, b.reshape(1, d))


def _gather_rows(tbl_ref, idx_ref, out_ref, base, nchunks, buf, sem):
    def chunk(c, carry):
        sl = idx_ref.at[pl.ds(c * 128, 128)]
        pltpu.async_copy(tbl_ref.at[sl], buf, sem).wait()
        pltpu.sync_copy(buf, out_ref.at[pl.ds(base + c * 128, 128)])
        return carry

    lax.fori_loop(0, nchunks, chunk, 0)


def _make_gather_kernel(B, T, D, NI, NU):
    n_rows = B * T
    per_w = n_rows // NUM_WORKERS
    n_chunks = per_w // 128
    u_per_w = B // NUM_WORKERS
    u_chunks = u_per_w // 128
    mesh = plsc.VectorSubcoreMesh(core_axis_name="c", subcore_axis_name="s")

    @functools.partial(
        pl.kernel,
        out_type=[
            jax.ShapeDtypeStruct((n_rows, D), jnp.float32),
            jax.ShapeDtypeStruct((B, D), jnp.float32),
        ],
        mesh=mesh,
        scratch_types=[
            pltpu.VMEM((per_w,), jnp.int32),
            pltpu.VMEM((u_per_w,), jnp.int32),
            pltpu.VMEM((128, D), jnp.float32),
            pltpu.SemaphoreType.DMA,
        ],
    )
    def gather_kernel(g_hbm, ut_hbm, nidx_hbm, nodes_hbm, nf_hbm, u_hbm,
                      idx_v, uidx_v, buf, sem):
        wid = lax.axis_index("s") * 2 + lax.axis_index("c")
        pltpu.sync_copy(nidx_hbm.at[pl.ds(wid * per_w, per_w)], idx_v)
        _gather_rows(g_hbm, idx_v, nf_hbm, wid * per_w, n_chunks, buf, sem)
        pltpu.sync_copy(nodes_hbm.at[pl.ds(wid * u_per_w, u_per_w)], uidx_v)
        _gather_rows(ut_hbm, uidx_v, u_hbm, wid * u_per_w, u_chunks, buf, sem)

    return gather_kernel


def _attn_body(T, BB, u_ref, nf3_ref, nfe_ref, wu_ref, bu_ref, w1a_ref,
               w1b_ref, b1_ref, w2_ref, fea_ref, e2_ref):
    pid = pl.program_id(0)

    @pl.when(pid == 0)
    def _zero_block():
        e2_ref[...] = jnp.zeros_like(e2_ref)

    @pl.when(pid > 0)
    def _compute():
        u = u_ref[...]
        nfea = (
            jnp.dot(u, wu_ref[...], preferred_element_type=jnp.float32)
            + bu_ref[...]
        )
        fea_ref[...] = nfea
        node_repr = nfea + nfe_ref[...]
        base = (
            jnp.dot(node_repr, w1b_ref[...], preferred_element_type=jnp.float32)
            + b1_ref[...]
        )
        w1a = w1a_ref[...]
        w2 = w2_ref[...]
        rows = []
        for t in range(T):
            ht = jnp.maximum(
                jnp.dot(nf3_ref[t], w1a, preferred_element_type=jnp.float32)
                + base,
                0.0,
            )
            rows.append(
                lax.dot_general(w2, ht, (((1,), (1,)), ((), ())),
                                preferred_element_type=jnp.float32)
            )
        logits = jnp.concatenate(rows, axis=0)
        m = jnp.max(logits, axis=0, keepdims=True)
        e = jnp.exp(logits - m)
        att = e / jnp.sum(e, axis=0, keepdims=True)
        eye = (
            lax.broadcasted_iota(jnp.int32, (T, T), 0)
            == lax.broadcasted_iota(jnp.int32, (T, T), 1)
        ).astype(jnp.float32)
        att_t = lax.dot_general(att, eye, (((0,), (0,)), ((), ())),
                                preferred_element_type=jnp.float32)
        acc = att_t[:, 0:1] * nf3_ref[0]
        for t in range(1, T):
            acc = acc + att_t[:, t:t + 1] * nf3_ref[t]
        e2_ref[...] = acc


def _attention(U, NF3, n_feature, Wu, bu, att_W1, att_b1, att_W2, BB):
    B, D = U.shape
    T = NF3.shape[0]
    nblk = B // BB
    grid = nblk + 1

    def shifted(i):
        return jnp.maximum(i - 1, 0)

    body = functools.partial(_attn_body, T, BB)
    return pl.pallas_call(
        body,
        grid=(grid,),
        in_specs=[
            pl.BlockSpec((BB, D), lambda i: (shifted(i), 0)),
            pl.BlockSpec((T, BB, D), lambda i: (0, shifted(i), 0)),
            pl.BlockSpec((BB, D), lambda i: (shifted(i), 0)),
            pl.BlockSpec((D, D), lambda i: (0, 0)),
            pl.BlockSpec((1, D), lambda i: (0, 0)),
            pl.BlockSpec((D, D), lambda i: (0, 0)),
            pl.BlockSpec((D, D), lambda i: (0, 0)),
            pl.BlockSpec((1, D), lambda i: (0, 0)),
            pl.BlockSpec((1, D), lambda i: (0, 0)),
        ],
        out_specs=[
            pl.BlockSpec((BB, D), lambda i: (shifted(i), 0)),
            pl.BlockSpec((BB, D), lambda i: (i, 0)),
        ],
        out_shape=[
            jax.ShapeDtypeStruct((B, D), jnp.float32),
            jax.ShapeDtypeStruct((B + BB, D), jnp.float32),
        ],
    )(
        U, NF3, n_feature, Wu, bu.reshape(1, D),
        att_W1[:D], att_W1[D:], att_b1.reshape(1, D),
        att_W2.reshape(1, D),
    )


def _make_scatter_kernel(B, D, NU, BB):
    n_chunks = NU // 128
    tail = NU - n_chunks * 128
    max_lc = n_chunks // NUM_WORKERS + 1
    slab = max_lc * 128
    n_batches = B // LANES
    init_steps = slab // LANES
    mesh = plsc.VectorSubcoreMesh(core_axis_name="c", subcore_axis_name="s")

    @functools.partial(
        pl.kernel,
        out_type=jax.ShapeDtypeStruct((NU, D), jnp.float32),
        mesh=mesh,
        scratch_types=[
            pltpu.VMEM((B,), jnp.int32),
            pltpu.VMEM((slab,), jnp.int32),
            pltpu.VMEM((128, D), jnp.float32),
            pltpu.SemaphoreType.DMA,
        ],
        compiler_params=pltpu.CompilerParams(needs_layout_passes=False),
    )
    def scatter_kernel(nodes_hbm, e2_hbm, out_hbm, nodes_v, slot_v, buf, sem):
        wid = lax.axis_index("s") * 2 + lax.axis_index("c")
        pltpu.sync_copy(nodes_hbm, nodes_v)

        def init(i, carry):
            slot_v[pl.ds(i * LANES, LANES)] = jnp.zeros((LANES,), jnp.int32)
            return carry

        lax.fori_loop(0, init_steps, init, 0)

        lane = lax.iota(jnp.int32, LANES)

        def build(b, carry):
            nv = nodes_v[pl.ds(b * LANES, LANES)]
            mask = ((nv >> 7) & (NUM_WORKERS - 1)) == wid
            local = ((nv >> 12) << 7) | (nv & 127)
            vals = b * LANES + lane + BB
            plsc.store_scatter(slot_v, [local], vals, mask=mask)
            return carry

        lax.fori_loop(0, n_batches, build, 0)

        def emit(lc, carry):
            gc = lc * NUM_WORKERS + wid

            @pl.when(gc < n_chunks)
            def _full():
                sl = slot_v.at[pl.ds(lc * 128, 128)]
                pltpu.async_copy(e2_hbm.at[sl], buf, sem).wait()
                pltpu.sync_copy(buf, out_hbm.at[pl.ds(gc * 128, 128)])

            @pl.when(gc == n_chunks)
            def _tail():
                sl = slot_v.at[pl.ds(lc * 128, tail)]
                bt = buf.at[pl.ds(0, tail)]
                pltpu.async_copy(e2_hbm.at[sl], bt, sem).wait()
                pltpu.sync_copy(bt, out_hbm.at[pl.ds(gc * 128, tail)])

            return carry

        lax.fori_loop(0, max_lc, emit, 0)

    return scatter_kernel


def kernel(nodes, n_feature, neigh_idx, u_table, i_table,
           Wu, bu, Wi, bi, att_W1, att_b1, att_W2, att_b2):
    B, T = neigh_idx.shape
    NU, DF = u_table.shape
    NI = i_table.shape[0]
    D = Wu.shape[1]
    BB = 512

    G = _row_transform(i_table, Wi, bi, 1000)

    nidx_t = neigh_idx.T.reshape(-1)
    NF, U = _make_gather_kernel(B, T, D, NI, NU)(G, u_table, nidx_t, nodes)
    NF3 = NF.reshape(T, B, D)

    nodes_fea, E2 = _attention(U, NF3, n_feature, Wu, bu,
                               att_W1, att_b1, att_W2, BB)

    embed_matrix = _make_scatter_kernel(B, D, NU, BB)(nodes, E2)

    return (nodes_fea, embed_matrix)


# speedup vs baseline: 5.7184x; 5.7184x over previous
import functools

import jax
import jax.numpy as jnp
from jax import lax
from jax.experimental import pallas as pl
from jax.experimental.pallas import tpu as pltpu
from jax.experimental.pallas import tpu_sc as plsc

NUM_WORKERS = 32
LANES = 16


def _row_transform_body(x_ref, w_ref, b_ref, o_ref):
    o_ref[...] = (
        jnp.dot(x_ref[...], w_ref[...], preferred_element_type=jnp.float32)
        + b_ref[...]
    )


def _row_transform(table, W, b, block_rows):
    n, df = table.shape
    d = W.shape[1]
    grid = n // block_rows
    return pl.pallas_call(
        _row_transform_body,
        grid=(grid,),
        in_specs=[
            pl.BlockSpec((block_rows, df), lambda i: (i, 0)),
            pl.BlockSpec((df, d), lambda i: (0, 0)),
            pl.BlockSpec((1, d), lambda i: (0, 0)),
        ],
        out_specs=pl.BlockSpec((block_rows, d), lambda i: (i, 0)),
        out_shape=jax.ShapeDtypeStruct((n, d), jnp.float32),
    )(table, W, b.reshape(1, d))


def _gather_rows(tbl_ref, idx_ref, out_ref, base, nchunks, buf, sem):
    def chunk(c, carry):
        sl = idx_ref.at[pl.ds(c * 128, 128)]
        pltpu.async_copy(tbl_ref.at[sl], buf, sem).wait()
        pltpu.sync_copy(buf, out_ref.at[pl.ds(base + c * 128, 128)])
        return carry

    lax.fori_loop(0, nchunks, chunk, 0)


def _make_gather_kernel(B, T, D, NI, NU):
    n_rows = B * T
    per_w = n_rows // NUM_WORKERS
    n_chunks = per_w // 128
    u_per_w = B // NUM_WORKERS
    u_chunks = u_per_w // 128
    mesh = plsc.VectorSubcoreMesh(core_axis_name="c", subcore_axis_name="s")

    @functools.partial(
        pl.kernel,
        out_type=[
            jax.ShapeDtypeStruct((n_rows, D), jnp.float32),
            jax.ShapeDtypeStruct((B, D), jnp.float32),
        ],
        mesh=mesh,
        scratch_types=[
            pltpu.VMEM((per_w,), jnp.int32),
            pltpu.VMEM((u_per_w,), jnp.int32),
            pltpu.VMEM((128, D), jnp.float32),
            pltpu.SemaphoreType.DMA,
        ],
    )
    def gather_kernel(g_hbm, ut_hbm, nidx_hbm, nodes_hbm, nf_hbm, u_hbm,
                      idx_v, uidx_v, buf, sem):
        wid = lax.axis_index("s") * 2 + lax.axis_index("c")
        pltpu.sync_copy(nidx_hbm.at[pl.ds(wid * per_w, per_w)], idx_v)
        _gather_rows(g_hbm, idx_v, nf_hbm, wid * per_w, n_chunks, buf, sem)
        pltpu.sync_copy(nodes_hbm.at[pl.ds(wid * u_per_w, u_per_w)], uidx_v)
        _gather_rows(ut_hbm, uidx_v, u_hbm, wid * u_per_w, u_chunks, buf, sem)

    return gather_kernel


def _attn_body(T, BB, u_ref, nf3_ref, nfe_ref, wu_ref, bu_ref, w1a_ref,
               w1b_ref, b1_ref, w2_ref, fea_ref, e2_ref):
    pid = pl.program_id(0)

    @pl.when(pid == 0)
    def _zero_block():
        e2_ref[...] = jnp.zeros_like(e2_ref)

    @pl.when(pid > 0)
    def _compute():
        u = u_ref[...]
        nfea = (
            jnp.dot(u, wu_ref[...], preferred_element_type=jnp.float32)
            + bu_ref[...]
        )
        fea_ref[...] = nfea
        node_repr = nfea + nfe_ref[...]
        base = (
            jnp.dot(node_repr, w1b_ref[...], preferred_element_type=jnp.float32)
            + b1_ref[...]
        )
        w1a = w1a_ref[...]
        w2 = w2_ref[...]
        rows = []
        for t in range(T):
            ht = jnp.maximum(
                jnp.dot(nf3_ref[t], w1a, preferred_element_type=jnp.float32)
                + base,
                0.0,
            )
            rows.append(
                lax.dot_general(w2, ht, (((1,), (1,)), ((), ())),
                                preferred_element_type=jnp.float32)
            )
        logits = jnp.concatenate(rows, axis=0)
        m = jnp.max(logits, axis=0, keepdims=True)
        e = jnp.exp(logits - m)
        att = e / jnp.sum(e, axis=0, keepdims=True)
        eye = (
            lax.broadcasted_iota(jnp.int32, (T, T), 0)
            == lax.broadcasted_iota(jnp.int32, (T, T), 1)
        ).astype(jnp.float32)
        att_t = lax.dot_general(att, eye, (((0,), (0,)), ((), ())),
                                preferred_element_type=jnp.float32)
        acc = att_t[:, 0:1] * nf3_ref[0]
        for t in range(1, T):
            acc = acc + att_t[:, t:t + 1] * nf3_ref[t]
        e2_ref[...] = acc


def _attention(U, NF3, n_feature, Wu, bu, att_W1, att_b1, att_W2, BB):
    B, D = U.shape
    T = NF3.shape[0]
    nblk = B // BB
    grid = nblk + 1

    def shifted(i):
        return jnp.maximum(i - 1, 0)

    body = functools.partial(_attn_body, T, BB)
    return pl.pallas_call(
        body,
        grid=(grid,),
        in_specs=[
            pl.BlockSpec((BB, D), lambda i: (shifted(i), 0)),
            pl.BlockSpec((T, BB, D), lambda i: (0, shifted(i), 0)),
            pl.BlockSpec((BB, D), lambda i: (shifted(i), 0)),
            pl.BlockSpec((D, D), lambda i: (0, 0)),
            pl.BlockSpec((1, D), lambda i: (0, 0)),
            pl.BlockSpec((D, D), lambda i: (0, 0)),
            pl.BlockSpec((D, D), lambda i: (0, 0)),
            pl.BlockSpec((1, D), lambda i: (0, 0)),
            pl.BlockSpec((1, D), lambda i: (0, 0)),
        ],
        out_specs=[
            pl.BlockSpec((BB, D), lambda i: (shifted(i), 0)),
            pl.BlockSpec((BB, D), lambda i: (i, 0)),
        ],
        out_shape=[
            jax.ShapeDtypeStruct((B, D), jnp.float32),
            jax.ShapeDtypeStruct((B + BB, D), jnp.float32),
        ],
    )(
        U, NF3, n_feature, Wu, bu.reshape(1, D),
        att_W1[:D], att_W1[D:], att_b1.reshape(1, D),
        att_W2.reshape(1, D),
    )


def _make_scatter_kernel(B, D, NU, BB):
    n_chunks = NU // 128
    tail = NU - n_chunks * 128
    max_lc = n_chunks // NUM_WORKERS + 1
    slab = max_lc * 128
    n_batches = B // LANES
    init_steps = slab // LANES
    mesh = plsc.VectorSubcoreMesh(core_axis_name="c", subcore_axis_name="s")

    @functools.partial(
        pl.kernel,
        out_type=jax.ShapeDtypeStruct((NU, D), jnp.float32),
        mesh=mesh,
        scratch_types=[
            pltpu.VMEM((B,), jnp.int32),
            pltpu.VMEM((slab,), jnp.int32),
            pltpu.VMEM((128, D), jnp.float32),
            pltpu.SemaphoreType.DMA,
        ],
        compiler_params=pltpu.CompilerParams(needs_layout_passes=False),
    )
    def scatter_kernel(nodes_hbm, e2_hbm, out_hbm, nodes_v, slot_v, buf, sem):
        wid = lax.axis_index("s") * 2 + lax.axis_index("c")
        pltpu.sync_copy(nodes_hbm, nodes_v)

        lane = lax.iota(jnp.int32, LANES)

        def init(i, carry):
            slot_v[pl.ds(i * LANES, LANES)] = (i * LANES + lane) & (BB - 1)
            return carry

        lax.fori_loop(0, init_steps, init, 0)

        def build(b, carry):
            nv = nodes_v[pl.ds(b * LANES, LANES)]
            mask = ((nv >> 7) & (NUM_WORKERS - 1)) == wid
            local = ((nv >> 12) << 7) | (nv & 127)
            vals = b * LANES + lane + BB
            plsc.store_scatter(slot_v, [local], vals, mask=mask)
            return carry

        lax.fori_loop(0, n_batches, build, 0)

        def emit(lc, carry):
            gc = lc * NUM_WORKERS + wid

            @pl.when(gc < n_chunks)
            def _full():
                sl = slot_v.at[pl.ds(lc * 128, 128)]
                pltpu.async_copy(e2_hbm.at[sl], buf, sem).wait()
                pltpu.sync_copy(buf, out_hbm.at[pl.ds(gc * 128, 128)])

            @pl.when(gc == n_chunks)
            def _tail():
                sl = slot_v.at[pl.ds(lc * 128, tail)]
                bt = buf.at[pl.ds(0, tail)]
                pltpu.async_copy(e2_hbm.at[sl], bt, sem).wait()
                pltpu.sync_copy(bt, out_hbm.at[pl.ds(gc * 128, tail)])

            return carry

        lax.fori_loop(0, max_lc, emit, 0)

    return scatter_kernel


def kernel(nodes, n_feature, neigh_idx, u_table, i_table,
           Wu, bu, Wi, bi, att_W1, att_b1, att_W2, att_b2):
    B, T = neigh_idx.shape
    NU, DF = u_table.shape
    NI = i_table.shape[0]
    D = Wu.shape[1]
    BB = 512

    G = _row_transform(i_table, Wi, bi, 1000)

    nidx_t = neigh_idx.T.reshape(-1)
    NF, U = _make_gather_kernel(B, T, D, NI, NU)(G, u_table, nidx_t, nodes)
    NF3 = NF.reshape(T, B, D)

    nodes_fea, E2 = _attention(U, NF3, n_feature, Wu, bu,
                               att_W1, att_b1, att_W2, BB)

    embed_matrix = _make_scatter_kernel(B, D, NU, BB)(nodes, E2)

    return (nodes_fea, embed_matrix)


# speedup vs baseline: 8.1628x; 1.4275x over previous
import functools

import jax
import jax.numpy as jnp
from jax import lax
from jax.experimental import pallas as pl
from jax.experimental.pallas import tpu as pltpu
from jax.experimental.pallas import tpu_sc as plsc

NUM_WORKERS = 32
LANES = 16


def _row_transform_body(x_ref, w_ref, b_ref, o_ref):
    o_ref[...] = (
        jnp.dot(x_ref[...], w_ref[...], preferred_element_type=jnp.float32)
        + b_ref[...]
    )


def _row_transform(table, W, b, block_rows):
    n, df = table.shape
    d = W.shape[1]
    grid = n // block_rows
    return pl.pallas_call(
        _row_transform_body,
        grid=(grid,),
        in_specs=[
            pl.BlockSpec((block_rows, df), lambda i: (i, 0)),
            pl.BlockSpec((df, d), lambda i: (0, 0)),
            pl.BlockSpec((1, d), lambda i: (0, 0)),
        ],
        out_specs=pl.BlockSpec((block_rows, d), lambda i: (i, 0)),
        out_shape=jax.ShapeDtypeStruct((n, d), jnp.float32),
    )(table, W, b.reshape(1, d))


def _gather_rows(tbl_ref, idx_ref, out_ref, base, nchunks, buf, sem):
    def chunk(c, carry):
        sl = idx_ref.at[pl.ds(c * 128, 128)]
        pltpu.async_copy(tbl_ref.at[sl], buf, sem).wait()
        pltpu.sync_copy(buf, out_ref.at[pl.ds(base + c * 128, 128)])
        return carry

    lax.fori_loop(0, nchunks, chunk, 0)


def _make_gather_kernel(B, T, D, NI, NU):
    n_rows = B * T
    per_w = n_rows // NUM_WORKERS
    n_chunks = per_w // 128
    u_per_w = B // NUM_WORKERS
    u_chunks = u_per_w // 128
    mesh = plsc.VectorSubcoreMesh(core_axis_name="c", subcore_axis_name="s")

    @functools.partial(
        pl.kernel,
        out_type=[
            jax.ShapeDtypeStruct((n_rows, D), jnp.float32),
            jax.ShapeDtypeStruct((B, D), jnp.float32),
        ],
        mesh=mesh,
        scratch_types=[
            pltpu.VMEM((per_w,), jnp.int32),
            pltpu.VMEM((u_per_w,), jnp.int32),
            pltpu.VMEM((128, D), jnp.float32),
            pltpu.SemaphoreType.DMA,
        ],
    )
    def gather_kernel(g_hbm, ut_hbm, nidx_hbm, nodes_hbm, nf_hbm, u_hbm,
                      idx_v, uidx_v, buf, sem):
        wid = lax.axis_index("s") * 2 + lax.axis_index("c")
        pltpu.sync_copy(nidx_hbm.at[pl.ds(wid * per_w, per_w)], idx_v)
        _gather_rows(g_hbm, idx_v, nf_hbm, wid * per_w, n_chunks, buf, sem)
        pltpu.sync_copy(nodes_hbm.at[pl.ds(wid * u_per_w, u_per_w)], uidx_v)
        _gather_rows(ut_hbm, uidx_v, u_hbm, wid * u_per_w, u_chunks, buf, sem)

    return gather_kernel


def _attn_body(T, BB, u_ref, nf3_ref, nfe_ref, wu_ref, bu_ref, wi_ref,
               bi_ref, w1a_ref, w1b_ref, b1_ref, w2_ref, fea_ref, e2_ref):
    pid = pl.program_id(0)
    D = u_ref.shape[1]

    @pl.when(pid == 0)
    def _zero_block():
        e2_ref[...] = jnp.zeros_like(e2_ref)

    @pl.when(pid > 0)
    def _compute():
        u = u_ref[...]
        wi = wi_ref[...]
        bi = bi_ref[...]
        w1a = w1a_ref[...]
        nfea = (
            jnp.dot(u, wu_ref[...], preferred_element_type=jnp.float32)
            + bu_ref[...]
        )
        fea_ref[...] = nfea
        node_repr = nfea + nfe_ref[...]
        base = (
            jnp.dot(node_repr, w1b_ref[...], preferred_element_type=jnp.float32)
            + b1_ref[...]
            + jnp.dot(bi, w1a, preferred_element_type=jnp.float32)
        )
        w1a_eff = jnp.dot(wi, w1a, preferred_element_type=jnp.float32)
        w2 = w2_ref[...]
        nf_flat = nf3_ref[...].reshape(T * BB, D)
        base_rep = jnp.broadcast_to(base[None], (T, BB, D)).reshape(T * BB, D)
        h = jnp.maximum(
            jnp.dot(nf_flat, w1a_eff, preferred_element_type=jnp.float32)
            + base_rep,
            0.0,
        )
        rows = []
        for t in range(T):
            rows.append(
                lax.dot_general(w2, h[t * BB:(t + 1) * BB],
                                (((1,), (1,)), ((), ())),
                                preferred_element_type=jnp.float32)
            )
        logits = jnp.concatenate(rows, axis=0)
        m = jnp.max(logits, axis=0, keepdims=True)
        e = jnp.exp(logits - m)
        att = e / jnp.sum(e, axis=0, keepdims=True)
        eye = (
            lax.broadcasted_iota(jnp.int32, (T, T), 0)
            == lax.broadcasted_iota(jnp.int32, (T, T), 1)
        ).astype(jnp.float32)
        att_t = lax.dot_general(att, eye, (((0,), (0,)), ((), ())),
                                preferred_element_type=jnp.float32)
        acc = att_t[:, 0:1] * nf3_ref[0]
        for t in range(1, T):
            acc = acc + att_t[:, t:t + 1] * nf3_ref[t]
        e2_ref[...] = (
            jnp.dot(acc, wi, preferred_element_type=jnp.float32) + bi
        )


def _attention(U, NF3, n_feature, Wu, bu, Wi, bi, att_W1, att_b1, att_W2, BB):
    B, D = U.shape
    T = NF3.shape[0]
    nblk = B // BB
    grid = nblk + 1

    def shifted(i):
        return jnp.maximum(i - 1, 0)

    body = functools.partial(_attn_body, T, BB)
    return pl.pallas_call(
        body,
        grid=(grid,),
        in_specs=[
            pl.BlockSpec((BB, D), lambda i: (shifted(i), 0)),
            pl.BlockSpec((T, BB, D), lambda i: (0, shifted(i), 0)),
            pl.BlockSpec((BB, D), lambda i: (shifted(i), 0)),
            pl.BlockSpec((D, D), lambda i: (0, 0)),
            pl.BlockSpec((1, D), lambda i: (0, 0)),
            pl.BlockSpec((D, D), lambda i: (0, 0)),
            pl.BlockSpec((1, D), lambda i: (0, 0)),
            pl.BlockSpec((D, D), lambda i: (0, 0)),
            pl.BlockSpec((D, D), lambda i: (0, 0)),
            pl.BlockSpec((1, D), lambda i: (0, 0)),
            pl.BlockSpec((1, D), lambda i: (0, 0)),
        ],
        out_specs=[
            pl.BlockSpec((BB, D), lambda i: (shifted(i), 0)),
            pl.BlockSpec((BB, D), lambda i: (i, 0)),
        ],
        out_shape=[
            jax.ShapeDtypeStruct((B, D), jnp.float32),
            jax.ShapeDtypeStruct((B + BB, D), jnp.float32),
        ],
    )(
        U, NF3, n_feature, Wu, bu.reshape(1, D),
        Wi, bi.reshape(1, D),
        att_W1[:D], att_W1[D:], att_b1.reshape(1, D),
        att_W2.reshape(1, D),
    )


def _make_scatter_kernel(B, D, NU, BB):
    n_chunks = NU // 128
    tail = NU - n_chunks * 128
    max_lc = n_chunks // NUM_WORKERS + 1
    slab = max_lc * 128
    n_batches = B // LANES
    init_steps = slab // LANES
    mesh = plsc.VectorSubcoreMesh(core_axis_name="c", subcore_axis_name="s")

    @functools.partial(
        pl.kernel,
        out_type=jax.ShapeDtypeStruct((NU, D), jnp.float32),
        mesh=mesh,
        scratch_types=[
            pltpu.VMEM((B,), jnp.int32),
            pltpu.VMEM((slab,), jnp.int32),
            pltpu.VMEM((128, D), jnp.float32),
            pltpu.SemaphoreType.DMA,
        ],
        compiler_params=pltpu.CompilerParams(needs_layout_passes=False),
    )
    def scatter_kernel(nodes_hbm, e2_hbm, out_hbm, nodes_v, slot_v, buf, sem):
        wid = lax.axis_index("s") * 2 + lax.axis_index("c")
        pltpu.sync_copy(nodes_hbm, nodes_v)

        lane = lax.iota(jnp.int32, LANES)

        def init(i, carry):
            slot_v[pl.ds(i * LANES, LANES)] = (i * LANES + lane) & (BB - 1)
            return carry

        lax.fori_loop(0, init_steps, init, 0)

        def build(b, carry):
            nv = nodes_v[pl.ds(b * LANES, LANES)]
            mask = ((nv >> 7) & (NUM_WORKERS - 1)) == wid
            local = ((nv >> 12) << 7) | (nv & 127)
            vals = b * LANES + lane + BB
            plsc.store_scatter(slot_v, [local], vals, mask=mask)
            return carry

        lax.fori_loop(0, n_batches, build, 0)

        def emit(lc, carry):
            gc = lc * NUM_WORKERS + wid

            @pl.when(gc < n_chunks)
            def _full():
                sl = slot_v.at[pl.ds(lc * 128, 128)]
                pltpu.async_copy(e2_hbm.at[sl], buf, sem).wait()
                pltpu.sync_copy(buf, out_hbm.at[pl.ds(gc * 128, 128)])

            @pl.when(gc == n_chunks)
            def _tail():
                sl = slot_v.at[pl.ds(lc * 128, tail)]
                bt = buf.at[pl.ds(0, tail)]
                pltpu.async_copy(e2_hbm.at[sl], bt, sem).wait()
                pltpu.sync_copy(bt, out_hbm.at[pl.ds(gc * 128, tail)])

            return carry

        lax.fori_loop(0, max_lc, emit, 0)

    return scatter_kernel


def kernel(nodes, n_feature, neigh_idx, u_table, i_table,
           Wu, bu, Wi, bi, att_W1, att_b1, att_W2, att_b2):
    B, T = neigh_idx.shape
    NU, DF = u_table.shape
    NI = i_table.shape[0]
    D = Wu.shape[1]
    BB = 512

    nidx_t = neigh_idx.T.reshape(-1)
    NF, U = _make_gather_kernel(B, T, D, NI, NU)(
        i_table, u_table, nidx_t, nodes)
    NF3 = NF.reshape(T, B, D)

    nodes_fea, E2 = _attention(U, NF3, n_feature, Wu, bu, Wi, bi,
                               att_W1, att_b1, att_W2, BB)

    embed_matrix = _make_scatter_kernel(B, D, NU, BB)(nodes, E2)

    return (nodes_fea, embed_matrix)


# speedup vs baseline: 9.2454x; 1.1326x over previous
import functools

import jax
import jax.numpy as jnp
from jax import lax
from jax.experimental import pallas as pl
from jax.experimental.pallas import tpu as pltpu
from jax.experimental.pallas import tpu_sc as plsc

NUM_WORKERS = 32
LANES = 16


def _row_transform_body(x_ref, w_ref, b_ref, o_ref):
    o_ref[...] = (
        jnp.dot(x_ref[...], w_ref[...], preferred_element_type=jnp.float32)
        + b_ref[...]
    )


def _row_transform(table, W, b, block_rows):
    n, df = table.shape
    d = W.shape[1]
    grid = n // block_rows
    return pl.pallas_call(
        _row_transform_body,
        grid=(grid,),
        in_specs=[
            pl.BlockSpec((block_rows, df), lambda i: (i, 0)),
            pl.BlockSpec((df, d), lambda i: (0, 0)),
            pl.BlockSpec((1, d), lambda i: (0, 0)),
        ],
        out_specs=pl.BlockSpec((block_rows, d), lambda i: (i, 0)),
        out_shape=jax.ShapeDtypeStruct((n, d), jnp.float32),
    )(table, W, b.reshape(1, d))


def _gather_rows(tbl_ref, idx_ref, out_ref, base, nchunks, buf, sem):
    def chunk(c, carry):
        sl = idx_ref.at[pl.ds(c * 128, 128)]
        pltpu.async_copy(tbl_ref.at[sl], buf, sem).wait()
        pltpu.sync_copy(buf, out_ref.at[pl.ds(base + c * 128, 128)])
        return carry

    lax.fori_loop(0, nchunks, chunk, 0)


def _gather_rows_pipelined(tbl_ref, idx_ref, out_ref, base, nchunks,
                           buf0, buf1, gsem0, gsem1, wsem0, wsem1):
    def fire_g(c, buf, sem):
        pltpu.async_copy(tbl_ref.at[idx_ref.at[pl.ds(c * 128, 128)]],
                         buf, sem)

    def wait_g(buf, sem):
        pltpu.make_async_copy(tbl_ref.at[pl.ds(0, 128)], buf, sem).wait()

    def fire_w(c, buf, sem):
        pltpu.async_copy(buf, out_ref.at[pl.ds(base + c * 128, 128)], sem)

    def wait_w(c, buf, sem):
        pltpu.make_async_copy(buf, out_ref.at[pl.ds(base + c * 128, 128)],
                              sem).wait()

    fire_g(0, buf0, gsem0)
    fire_g(1, buf1, gsem1)

    def body(i, carry):
        c = i * 2
        wait_g(buf0, gsem0)
        fire_w(c, buf0, wsem0)
        wait_g(buf1, gsem1)
        fire_w(c + 1, buf1, wsem1)

        @pl.when(c + 2 < nchunks)
        def _next0():
            wait_w(c, buf0, wsem0)
            fire_g(c + 2, buf0, gsem0)

        @pl.when(c + 3 < nchunks)
        def _next1():
            wait_w(c + 1, buf1, wsem1)
            fire_g(c + 3, buf1, gsem1)

        @pl.when(c + 2 >= nchunks)
        def _drain0():
            wait_w(c, buf0, wsem0)

        @pl.when(c + 3 >= nchunks)
        def _drain1():
            wait_w(c + 1, buf1, wsem1)

        return carry

    lax.fori_loop(0, nchunks // 2, body, 0, unroll=False)


def _make_gather_kernel(B, T, D, NI, NU):
    n_rows = B * T
    per_w = n_rows // NUM_WORKERS
    n_chunks = per_w // 128
    u_per_w = B // NUM_WORKERS
    u_chunks = u_per_w // 128
    mesh = plsc.VectorSubcoreMesh(core_axis_name="c", subcore_axis_name="s")

    @functools.partial(
        pl.kernel,
        out_type=[
            jax.ShapeDtypeStruct((n_rows, D), jnp.float32),
            jax.ShapeDtypeStruct((B, D), jnp.float32),
        ],
        mesh=mesh,
        scratch_types=[
            pltpu.VMEM((per_w,), jnp.int32),
            pltpu.VMEM((u_per_w,), jnp.int32),
            pltpu.VMEM((128, D), jnp.float32),
            pltpu.VMEM((128, D), jnp.float32),
            pltpu.SemaphoreType.DMA,
            pltpu.SemaphoreType.DMA,
            pltpu.SemaphoreType.DMA,
            pltpu.SemaphoreType.DMA,
        ],
    )
    def gather_kernel(g_hbm, ut_hbm, nidx_hbm, nodes_hbm, nf_hbm, u_hbm,
                      idx_v, uidx_v, buf0, buf1, gs0, gs1, ws0, ws1):
        wid = lax.axis_index("s") * 2 + lax.axis_index("c")
        pltpu.sync_copy(nidx_hbm.at[pl.ds(wid * per_w, per_w)], idx_v)
        _gather_rows_pipelined(g_hbm, idx_v, nf_hbm, wid * per_w, n_chunks,
                               buf0, buf1, gs0, gs1, ws0, ws1)
        pltpu.sync_copy(nodes_hbm.at[pl.ds(wid * u_per_w, u_per_w)], uidx_v)
        _gather_rows_pipelined(ut_hbm, uidx_v, u_hbm, wid * u_per_w, u_chunks,
                               buf0, buf1, gs0, gs1, ws0, ws1)

    return gather_kernel


def _attn_body(T, BB, u_ref, nf3_ref, nfe_ref, wu_ref, bu_ref, wi_ref,
               bi_ref, w1a_ref, w1b_ref, b1_ref, w2_ref, fea_ref, e2_ref):
    pid = pl.program_id(0)
    D = u_ref.shape[1]

    @pl.when(pid == 0)
    def _zero_block():
        e2_ref[...] = jnp.zeros_like(e2_ref)

    @pl.when(pid > 0)
    def _compute():
        u = u_ref[...]
        wi = wi_ref[...]
        bi = bi_ref[...]
        w1a = w1a_ref[...]
        nfea = (
            jnp.dot(u, wu_ref[...], preferred_element_type=jnp.float32)
            + bu_ref[...]
        )
        fea_ref[...] = nfea
        node_repr = nfea + nfe_ref[...]
        base = (
            jnp.dot(node_repr, w1b_ref[...], preferred_element_type=jnp.float32)
            + b1_ref[...]
            + jnp.dot(bi, w1a, preferred_element_type=jnp.float32)
        )
        w1a_eff = jnp.dot(wi, w1a, preferred_element_type=jnp.float32)
        w2 = w2_ref[...]
        nf_flat = nf3_ref[...].reshape(T * BB, D)
        base_rep = jnp.broadcast_to(base[None], (T, BB, D)).reshape(T * BB, D)
        h = jnp.maximum(
            jnp.dot(nf_flat, w1a_eff, preferred_element_type=jnp.float32)
            + base_rep,
            0.0,
        )
        rows = []
        for t in range(T):
            rows.append(
                lax.dot_general(w2, h[t * BB:(t + 1) * BB],
                                (((1,), (1,)), ((), ())),
                                preferred_element_type=jnp.float32)
            )
        logits = jnp.concatenate(rows, axis=0)
        m = jnp.max(logits, axis=0, keepdims=True)
        e = jnp.exp(logits - m)
        att = e / jnp.sum(e, axis=0, keepdims=True)
        eye = (
            lax.broadcasted_iota(jnp.int32, (T, T), 0)
            == lax.broadcasted_iota(jnp.int32, (T, T), 1)
        ).astype(jnp.float32)
        att_t = lax.dot_general(att, eye, (((0,), (0,)), ((), ())),
                                preferred_element_type=jnp.float32)
        acc = att_t[:, 0:1] * nf3_ref[0]
        for t in range(1, T):
            acc = acc + att_t[:, t:t + 1] * nf3_ref[t]
        e2_ref[...] = (
            jnp.dot(acc, wi, preferred_element_type=jnp.float32) + bi
        )


def _attention(U, NF3, n_feature, Wu, bu, Wi, bi, att_W1, att_b1, att_W2, BB):
    B, D = U.shape
    T = NF3.shape[0]
    nblk = B // BB
    grid = nblk + 1

    def shifted(i):
        return jnp.maximum(i - 1, 0)

    body = functools.partial(_attn_body, T, BB)
    return pl.pallas_call(
        body,
        grid=(grid,),
        in_specs=[
            pl.BlockSpec((BB, D), lambda i: (shifted(i), 0)),
            pl.BlockSpec((T, BB, D), lambda i: (0, shifted(i), 0)),
            pl.BlockSpec((BB, D), lambda i: (shifted(i), 0)),
            pl.BlockSpec((D, D), lambda i: (0, 0)),
            pl.BlockSpec((1, D), lambda i: (0, 0)),
            pl.BlockSpec((D, D), lambda i: (0, 0)),
            pl.BlockSpec((1, D), lambda i: (0, 0)),
            pl.BlockSpec((D, D), lambda i: (0, 0)),
            pl.BlockSpec((D, D), lambda i: (0, 0)),
            pl.BlockSpec((1, D), lambda i: (0, 0)),
            pl.BlockSpec((1, D), lambda i: (0, 0)),
        ],
        out_specs=[
            pl.BlockSpec((BB, D), lambda i: (shifted(i), 0)),
            pl.BlockSpec((BB, D), lambda i: (i, 0)),
        ],
        out_shape=[
            jax.ShapeDtypeStruct((B, D), jnp.float32),
            jax.ShapeDtypeStruct((B + BB, D), jnp.float32),
        ],
    )(
        U, NF3, n_feature, Wu, bu.reshape(1, D),
        Wi, bi.reshape(1, D),
        att_W1[:D], att_W1[D:], att_b1.reshape(1, D),
        att_W2.reshape(1, D),
    )


def _make_scatter_kernel(B, D, NU, BB):
    n_chunks = NU // 128
    tail = NU - n_chunks * 128
    max_lc = n_chunks // NUM_WORKERS + 1
    slab = max_lc * 128
    n_batches = B // LANES
    init_steps = slab // LANES
    mesh = plsc.VectorSubcoreMesh(core_axis_name="c", subcore_axis_name="s")

    @functools.partial(
        pl.kernel,
        out_type=jax.ShapeDtypeStruct((NU, D), jnp.float32),
        mesh=mesh,
        scratch_types=[
            pltpu.VMEM((B,), jnp.int32),
            pltpu.VMEM((slab,), jnp.int32),
            pltpu.VMEM((128, D), jnp.float32),
            pltpu.SemaphoreType.DMA,
        ],
        compiler_params=pltpu.CompilerParams(needs_layout_passes=False),
    )
    def scatter_kernel(nodes_hbm, e2_hbm, out_hbm, nodes_v, slot_v, buf, sem):
        wid = lax.axis_index("s") * 2 + lax.axis_index("c")
        pltpu.sync_copy(nodes_hbm, nodes_v)

        lane = lax.iota(jnp.int32, LANES)

        def init(i, carry):
            slot_v[pl.ds(i * LANES, LANES)] = (i * LANES + lane) & (BB - 1)
            return carry

        lax.fori_loop(0, init_steps, init, 0)

        def build(b, carry):
            nv = nodes_v[pl.ds(b * LANES, LANES)]
            mask = ((nv >> 7) & (NUM_WORKERS - 1)) == wid
            local = ((nv >> 12) << 7) | (nv & 127)
            vals = b * LANES + lane + BB
            plsc.store_scatter(slot_v, [local], vals, mask=mask)
            return carry

        lax.fori_loop(0, n_batches, build, 0)

        def emit(lc, carry):
            gc = lc * NUM_WORKERS + wid

            @pl.when(gc < n_chunks)
            def _full():
                sl = slot_v.at[pl.ds(lc * 128, 128)]
                pltpu.async_copy(e2_hbm.at[sl], buf, sem).wait()
                pltpu.sync_copy(buf, out_hbm.at[pl.ds(gc * 128, 128)])

            @pl.when(gc == n_chunks)
            def _tail():
                sl = slot_v.at[pl.ds(lc * 128, tail)]
                bt = buf.at[pl.ds(0, tail)]
                pltpu.async_copy(e2_hbm.at[sl], bt, sem).wait()
                pltpu.sync_copy(bt, out_hbm.at[pl.ds(gc * 128, tail)])

            return carry

        lax.fori_loop(0, max_lc, emit, 0)

    return scatter_kernel


def kernel(nodes, n_feature, neigh_idx, u_table, i_table,
           Wu, bu, Wi, bi, att_W1, att_b1, att_W2, att_b2):
    B, T = neigh_idx.shape
    NU, DF = u_table.shape
    NI = i_table.shape[0]
    D = Wu.shape[1]
    BB = 512

    nidx_t = neigh_idx.T.reshape(-1)
    NF, U = _make_gather_kernel(B, T, D, NI, NU)(
        i_table, u_table, nidx_t, nodes)
    NF3 = NF.reshape(T, B, D)

    nodes_fea, E2 = _attention(U, NF3, n_feature, Wu, bu, Wi, bi,
                               att_W1, att_b1, att_W2, BB)

    embed_matrix = _make_scatter_kernel(B, D, NU, BB)(nodes, E2)

    return (nodes_fea, embed_matrix)


# speedup vs baseline: 9.7524x; 1.0548x over previous
import functools

import jax
import jax.numpy as jnp
from jax import lax
from jax.experimental import pallas as pl
from jax.experimental.pallas import tpu as pltpu
from jax.experimental.pallas import tpu_sc as plsc

NUM_WORKERS = 32
LANES = 16


def _row_transform_body(x_ref, w_ref, b_ref, o_ref):
    o_ref[...] = (
        jnp.dot(x_ref[...], w_ref[...], preferred_element_type=jnp.float32)
        + b_ref[...]
    )


def _row_transform(table, W, b, block_rows):
    n, df = table.shape
    d = W.shape[1]
    grid = n // block_rows
    return pl.pallas_call(
        _row_transform_body,
        grid=(grid,),
        in_specs=[
            pl.BlockSpec((block_rows, df), lambda i: (i, 0)),
            pl.BlockSpec((df, d), lambda i: (0, 0)),
            pl.BlockSpec((1, d), lambda i: (0, 0)),
        ],
        out_specs=pl.BlockSpec((block_rows, d), lambda i: (i, 0)),
        out_shape=jax.ShapeDtypeStruct((n, d), jnp.float32),
    )(table, W, b.reshape(1, d))


def _gather_rows(tbl_ref, idx_ref, out_ref, base, nchunks, buf, sem):
    def chunk(c, carry):
        sl = idx_ref.at[pl.ds(c * 128, 128)]
        pltpu.async_copy(tbl_ref.at[sl], buf, sem).wait()
        pltpu.sync_copy(buf, out_ref.at[pl.ds(base + c * 128, 128)])
        return carry

    lax.fori_loop(0, nchunks, chunk, 0)


def _gather_rows_pipelined(tbl_ref, idx_ref, out_ref, base, nchunks,
                           buf0, buf1, gsem0, gsem1, wsem0, wsem1):
    def fire_g(c, buf, sem):
        pltpu.async_copy(tbl_ref.at[idx_ref.at[pl.ds(c * 128, 128)]],
                         buf, sem)

    def wait_g(buf, sem):
        pltpu.make_async_copy(tbl_ref.at[pl.ds(0, 128)], buf, sem).wait()

    def fire_w(c, buf, sem):
        pltpu.async_copy(buf, out_ref.at[pl.ds(base + c * 128, 128)], sem)

    def wait_w(c, buf, sem):
        pltpu.make_async_copy(buf, out_ref.at[pl.ds(base + c * 128, 128)],
                              sem).wait()

    fire_g(0, buf0, gsem0)
    fire_g(1, buf1, gsem1)

    def body(i, carry):
        c = i * 2
        wait_g(buf0, gsem0)
        fire_w(c, buf0, wsem0)
        wait_g(buf1, gsem1)
        fire_w(c + 1, buf1, wsem1)

        @pl.when(c + 2 < nchunks)
        def _next0():
            wait_w(c, buf0, wsem0)
            fire_g(c + 2, buf0, gsem0)

        @pl.when(c + 3 < nchunks)
        def _next1():
            wait_w(c + 1, buf1, wsem1)
            fire_g(c + 3, buf1, gsem1)

        @pl.when(c + 2 >= nchunks)
        def _drain0():
            wait_w(c, buf0, wsem0)

        @pl.when(c + 3 >= nchunks)
        def _drain1():
            wait_w(c + 1, buf1, wsem1)

        return carry

    lax.fori_loop(0, nchunks // 2, body, 0, unroll=False)


def _make_gather_kernel(B, T, D, NI, NU):
    n_rows = B * T
    per_w = n_rows // NUM_WORKERS
    n_chunks = per_w // 128
    u_per_w = B // NUM_WORKERS
    u_chunks = u_per_w // 128
    mesh = plsc.VectorSubcoreMesh(core_axis_name="c", subcore_axis_name="s")

    @functools.partial(
        pl.kernel,
        out_type=[
            jax.ShapeDtypeStruct((n_rows, D), jnp.float32),
            jax.ShapeDtypeStruct((B, D), jnp.float32),
        ],
        mesh=mesh,
        scratch_types=[
            pltpu.VMEM((per_w,), jnp.int32),
            pltpu.VMEM((u_per_w,), jnp.int32),
            pltpu.VMEM((128, D), jnp.float32),
            pltpu.VMEM((128, D), jnp.float32),
            pltpu.SemaphoreType.DMA,
            pltpu.SemaphoreType.DMA,
            pltpu.SemaphoreType.DMA,
            pltpu.SemaphoreType.DMA,
        ],
    )
    def gather_kernel(g_hbm, ut_hbm, nidx_hbm, nodes_hbm, nf_hbm, u_hbm,
                      idx_v, uidx_v, buf0, buf1, gs0, gs1, ws0, ws1):
        wid = lax.axis_index("s") * 2 + lax.axis_index("c")
        pltpu.sync_copy(nidx_hbm.at[pl.ds(wid * per_w, per_w)], idx_v)
        _gather_rows_pipelined(g_hbm, idx_v, nf_hbm, wid * per_w, n_chunks,
                               buf0, buf1, gs0, gs1, ws0, ws1)
        pltpu.sync_copy(nodes_hbm.at[pl.ds(wid * u_per_w, u_per_w)], uidx_v)
        _gather_rows_pipelined(ut_hbm, uidx_v, u_hbm, wid * u_per_w, u_chunks,
                               buf0, buf1, gs0, gs1, ws0, ws1)

    return gather_kernel


def _attn_body(T, BB, u_ref, nf3_ref, nfe_ref, wu_ref, bu_ref, wi_ref,
               bi_ref, w1a_ref, w1b_ref, b1_ref, w2_ref, fea_ref, e2_ref):
    pid = pl.program_id(0)
    D = u_ref.shape[1]

    @pl.when(pid == 0)
    def _zero_block():
        e2_ref[...] = jnp.zeros_like(e2_ref)

    @pl.when(pid > 0)
    def _compute():
        u = u_ref[...]
        wi = wi_ref[...]
        bi = bi_ref[...]
        w1a = w1a_ref[...]
        nfea = (
            jnp.dot(u, wu_ref[...], preferred_element_type=jnp.float32)
            + bu_ref[...]
        )
        fea_ref[...] = nfea
        node_repr = nfea + nfe_ref[...]
        base = (
            jnp.dot(node_repr, w1b_ref[...], preferred_element_type=jnp.float32)
            + b1_ref[...]
            + jnp.dot(bi, w1a, preferred_element_type=jnp.float32)
        )
        w1a_eff = jnp.dot(wi, w1a, preferred_element_type=jnp.float32)
        w2 = w2_ref[...]
        nf_flat = nf3_ref[...].reshape(T * BB, D)
        base_rep = jnp.broadcast_to(base[None], (T, BB, D)).reshape(T * BB, D)
        h = jnp.maximum(
            jnp.dot(nf_flat, w1a_eff, preferred_element_type=jnp.float32)
            + base_rep,
            0.0,
        )
        rows = []
        for t in range(T):
            rows.append(
                lax.dot_general(w2, h[t * BB:(t + 1) * BB],
                                (((1,), (1,)), ((), ())),
                                preferred_element_type=jnp.float32)
            )
        logits = jnp.concatenate(rows, axis=0)
        m = jnp.max(logits, axis=0, keepdims=True)
        e = jnp.exp(logits - m)
        att = e / jnp.sum(e, axis=0, keepdims=True)
        eye = (
            lax.broadcasted_iota(jnp.int32, (T, T), 0)
            == lax.broadcasted_iota(jnp.int32, (T, T), 1)
        ).astype(jnp.float32)
        att_t = lax.dot_general(att, eye, (((0,), (0,)), ((), ())),
                                preferred_element_type=jnp.float32)
        acc = att_t[:, 0:1] * nf3_ref[0]
        for t in range(1, T):
            acc = acc + att_t[:, t:t + 1] * nf3_ref[t]
        e2_ref[...] = (
            jnp.dot(acc, wi, preferred_element_type=jnp.float32) + bi
        )


def _attention(U, NF3, n_feature, Wu, bu, Wi, bi, att_W1, att_b1, att_W2, BB):
    B, D = U.shape
    T = NF3.shape[0]
    nblk = B // BB
    grid = nblk + 1

    def shifted(i):
        return jnp.maximum(i - 1, 0)

    body = functools.partial(_attn_body, T, BB)
    return pl.pallas_call(
        body,
        grid=(grid,),
        in_specs=[
            pl.BlockSpec((BB, D), lambda i: (shifted(i), 0)),
            pl.BlockSpec((T, BB, D), lambda i: (0, shifted(i), 0)),
            pl.BlockSpec((BB, D), lambda i: (shifted(i), 0)),
            pl.BlockSpec((D, D), lambda i: (0, 0)),
            pl.BlockSpec((1, D), lambda i: (0, 0)),
            pl.BlockSpec((D, D), lambda i: (0, 0)),
            pl.BlockSpec((1, D), lambda i: (0, 0)),
            pl.BlockSpec((D, D), lambda i: (0, 0)),
            pl.BlockSpec((D, D), lambda i: (0, 0)),
            pl.BlockSpec((1, D), lambda i: (0, 0)),
            pl.BlockSpec((1, D), lambda i: (0, 0)),
        ],
        out_specs=[
            pl.BlockSpec((BB, D), lambda i: (shifted(i), 0)),
            pl.BlockSpec((BB, D), lambda i: (i, 0)),
        ],
        out_shape=[
            jax.ShapeDtypeStruct((B, D), jnp.float32),
            jax.ShapeDtypeStruct((B + BB, D), jnp.float32),
        ],
    )(
        U, NF3, n_feature, Wu, bu.reshape(1, D),
        Wi, bi.reshape(1, D),
        att_W1[:D], att_W1[D:], att_b1.reshape(1, D),
        att_W2.reshape(1, D),
    )


def _make_scatter_kernel(B, D, NU, BB, H, E2_ROWS):
    n_chunks = NU // 128
    tail = NU - n_chunks * 128
    max_lc = n_chunks // NUM_WORKERS + 1
    slab = max_lc * 128
    n_batches = B // LANES
    init_steps = slab // LANES
    mesh = plsc.VectorSubcoreMesh(core_axis_name="c", subcore_axis_name="s")

    @functools.partial(
        pl.kernel,
        out_type=jax.ShapeDtypeStruct((NU, D), jnp.float32),
        mesh=mesh,
        name="sc_scatter",
        scratch_types=[
            pltpu.VMEM((B,), jnp.int32),
            pltpu.VMEM((slab,), jnp.int32),
            pltpu.VMEM((128, D), jnp.float32),
            pltpu.SemaphoreType.DMA,
        ],
        compiler_params=pltpu.CompilerParams(needs_layout_passes=False),
    )
    def scatter_kernel(nodes_hbm, e2_hbm, out_hbm, nodes_v, slot_v, buf, sem):
        wid = lax.axis_index("s") * 2 + lax.axis_index("c")
        pltpu.sync_copy(nodes_hbm, nodes_v)

        lane = lax.iota(jnp.int32, LANES)

        def init(i, carry):
            slot_v[pl.ds(i * LANES, LANES)] = (i * LANES + lane) & (BB - 1)
            return carry

        lax.fori_loop(0, init_steps, init, 0)

        def build(b, carry):
            nv = nodes_v[pl.ds(b * LANES, LANES)]
            mask = ((nv >> 7) & (NUM_WORKERS - 1)) == wid
            local = ((nv >> 12) << 7) | (nv & 127)
            jv = b * LANES + lane
            vals = jv + BB + jnp.where(jv >= H, BB, 0)
            plsc.store_scatter(slot_v, [local], vals, mask=mask)
            return carry

        lax.fori_loop(0, n_batches, build, 0)

        def emit(lc, carry):
            gc = lc * NUM_WORKERS + wid

            @pl.when(gc < n_chunks)
            def _full():
                sl = slot_v.at[pl.ds(lc * 128, 128)]
                pltpu.async_copy(e2_hbm.at[sl], buf, sem).wait()
                pltpu.sync_copy(buf, out_hbm.at[pl.ds(gc * 128, 128)])

            @pl.when(gc == n_chunks)
            def _tail():
                sl = slot_v.at[pl.ds(lc * 128, tail)]
                bt = buf.at[pl.ds(0, tail)]
                pltpu.async_copy(e2_hbm.at[sl], bt, sem).wait()
                pltpu.sync_copy(bt, out_hbm.at[pl.ds(gc * 128, tail)])

            return carry

        lax.fori_loop(0, max_lc, emit, 0)

    return scatter_kernel


def kernel(nodes, n_feature, neigh_idx, u_table, i_table,
           Wu, bu, Wi, bi, att_W1, att_b1, att_W2, att_b2):
    B, T = neigh_idx.shape
    NU, DF = u_table.shape
    NI = i_table.shape[0]
    D = Wu.shape[1]
    BB = 512

    H = B // 2
    gather = _make_gather_kernel(H, T, D, NI, NU)
    feas, e2s = [], []
    for h in range(2):
        sl = slice(h * H, (h + 1) * H)
        nidx_t = neigh_idx[sl].T.reshape(-1)
        NF, U = gather(i_table, u_table, nidx_t, nodes[sl])
        fea_h, e2_h = _attention(U, NF.reshape(T, H, D), n_feature[sl],
                                 Wu, bu, Wi, bi, att_W1, att_b1, att_W2, BB)
        feas.append(fea_h)
        e2s.append(e2_h)
    nodes_fea = jnp.concatenate(feas, axis=0)
    E2 = jnp.concatenate(e2s, axis=0)

    embed_matrix = _make_scatter_kernel(B, D, NU, BB, H, E2.shape[0])(
        nodes, E2)

    return (nodes_fea, embed_matrix)


# speedup vs baseline: 10.1475x; 1.0405x over previous
import functools

import jax
import jax.numpy as jnp
from jax import lax
from jax.experimental import pallas as pl
from jax.experimental.pallas import tpu as pltpu
from jax.experimental.pallas import tpu_sc as plsc

NUM_WORKERS = 32
LANES = 16


def _row_transform_body(x_ref, w_ref, b_ref, o_ref):
    o_ref[...] = (
        jnp.dot(x_ref[...], w_ref[...], preferred_element_type=jnp.float32)
        + b_ref[...]
    )


def _row_transform(table, W, b, block_rows):
    n, df = table.shape
    d = W.shape[1]
    grid = n // block_rows
    return pl.pallas_call(
        _row_transform_body,
        grid=(grid,),
        in_specs=[
            pl.BlockSpec((block_rows, df), lambda i: (i, 0)),
            pl.BlockSpec((df, d), lambda i: (0, 0)),
            pl.BlockSpec((1, d), lambda i: (0, 0)),
        ],
        out_specs=pl.BlockSpec((block_rows, d), lambda i: (i, 0)),
        out_shape=jax.ShapeDtypeStruct((n, d), jnp.float32),
    )(table, W, b.reshape(1, d))


def _gather_rows(tbl_ref, idx_ref, out_ref, base, nchunks, buf, sem):
    def chunk(c, carry):
        sl = idx_ref.at[pl.ds(c * 128, 128)]
        pltpu.async_copy(tbl_ref.at[sl], buf, sem).wait()
        pltpu.sync_copy(buf, out_ref.at[pl.ds(base + c * 128, 128)])
        return carry

    lax.fori_loop(0, nchunks, chunk, 0)


def _gather_rows_pipelined(tbl_ref, idx_ref, out_ref, base, nchunks,
                           buf0, buf1, gsem0, gsem1, wsem0, wsem1,
                           out_pos=None):
    if out_pos is None:
        out_pos = lambda c: base + c * 128

    def fire_g(c, buf, sem):
        pltpu.async_copy(tbl_ref.at[idx_ref.at[pl.ds(c * 128, 128)]],
                         buf, sem)

    def wait_g(buf, sem):
        pltpu.make_async_copy(tbl_ref.at[pl.ds(0, 128)], buf, sem).wait()

    def fire_w(c, buf, sem):
        pltpu.async_copy(buf, out_ref.at[pl.ds(out_pos(c), 128)], sem)

    def wait_w(c, buf, sem):
        pltpu.make_async_copy(buf, out_ref.at[pl.ds(out_pos(c), 128)],
                              sem).wait()

    fire_g(0, buf0, gsem0)
    fire_g(1, buf1, gsem1)

    def body(i, carry):
        c = i * 2
        wait_g(buf0, gsem0)
        fire_w(c, buf0, wsem0)
        wait_g(buf1, gsem1)
        fire_w(c + 1, buf1, wsem1)

        @pl.when(c + 2 < nchunks)
        def _next0():
            wait_w(c, buf0, wsem0)
            fire_g(c + 2, buf0, gsem0)

        @pl.when(c + 3 < nchunks)
        def _next1():
            wait_w(c + 1, buf1, wsem1)
            fire_g(c + 3, buf1, gsem1)

        @pl.when(c + 2 >= nchunks)
        def _drain0():
            wait_w(c, buf0, wsem0)

        @pl.when(c + 3 >= nchunks)
        def _drain1():
            wait_w(c + 1, buf1, wsem1)

        return carry

    lax.fori_loop(0, nchunks // 2, body, 0, unroll=False)


def _make_gather_kernel(B, T, D, NI, NU):
    n_rows = B * T
    per_w = n_rows // NUM_WORKERS
    n_chunks = per_w // 128
    u_per_w = B // NUM_WORKERS
    u_chunks = u_per_w // 128
    mesh = plsc.VectorSubcoreMesh(core_axis_name="c", subcore_axis_name="s")

    @functools.partial(
        pl.kernel,
        out_type=[
            jax.ShapeDtypeStruct((n_rows, D), jnp.float32),
            jax.ShapeDtypeStruct((B, D), jnp.float32),
        ],
        mesh=mesh,
        scratch_types=[
            pltpu.VMEM((per_w,), jnp.int32),
            pltpu.VMEM((u_per_w,), jnp.int32),
            pltpu.VMEM((128, D), jnp.float32),
            pltpu.VMEM((128, D), jnp.float32),
            pltpu.SemaphoreType.DMA,
            pltpu.SemaphoreType.DMA,
            pltpu.SemaphoreType.DMA,
            pltpu.SemaphoreType.DMA,
        ],
    )
    def gather_kernel(g_hbm, ut_hbm, nidx_hbm, nodes_hbm, nf_hbm, u_hbm,
                      idx_v, uidx_v, buf0, buf1, gs0, gs1, ws0, ws1):
        wid = lax.axis_index("s") * 2 + lax.axis_index("c")
        pltpu.sync_copy(nidx_hbm.at[pl.ds(wid * per_w, per_w)], idx_v)
        _gather_rows_pipelined(g_hbm, idx_v, nf_hbm, wid * per_w, n_chunks,
                               buf0, buf1, gs0, gs1, ws0, ws1)
        pltpu.sync_copy(nodes_hbm.at[pl.ds(wid * u_per_w, u_per_w)], uidx_v)
        _gather_rows_pipelined(ut_hbm, uidx_v, u_hbm, wid * u_per_w, u_chunks,
                               buf0, buf1, gs0, gs1, ws0, ws1)

    return gather_kernel


def _attn_body(T, BB, u_ref, nf3_ref, nfe_ref, wu_ref, bu_ref, wi_ref,
               bi_ref, w1a_ref, w1b_ref, b1_ref, w2_ref, fea_ref, e2_ref):
    pid = pl.program_id(0)
    D = u_ref.shape[1]

    @pl.when(pid == 0)
    def _zero_block():
        e2_ref[...] = jnp.zeros_like(e2_ref)

    @pl.when(pid > 0)
    def _compute():
        u = u_ref[...]
        wi = wi_ref[...]
        bi = bi_ref[...]
        w1a = w1a_ref[...]
        nfea = (
            jnp.dot(u, wu_ref[...], preferred_element_type=jnp.float32)
            + bu_ref[...]
        )
        fea_ref[...] = nfea
        node_repr = nfea + nfe_ref[...]
        base = (
            jnp.dot(node_repr, w1b_ref[...], preferred_element_type=jnp.float32)
            + b1_ref[...]
            + jnp.dot(bi, w1a, preferred_element_type=jnp.float32)
        )
        w1a_eff = jnp.dot(wi, w1a, preferred_element_type=jnp.float32)
        w2 = w2_ref[...]
        nf_flat = nf3_ref[...].reshape(T * BB, D)
        base_rep = jnp.broadcast_to(base[None], (T, BB, D)).reshape(T * BB, D)
        h = jnp.maximum(
            jnp.dot(nf_flat, w1a_eff, preferred_element_type=jnp.float32)
            + base_rep,
            0.0,
        )
        rows = []
        for t in range(T):
            rows.append(
                lax.dot_general(w2, h[t * BB:(t + 1) * BB],
                                (((1,), (1,)), ((), ())),
                                preferred_element_type=jnp.float32)
            )
        logits = jnp.concatenate(rows, axis=0)
        m = jnp.max(logits, axis=0, keepdims=True)
        e = jnp.exp(logits - m)
        att = e / jnp.sum(e, axis=0, keepdims=True)
        eye = (
            lax.broadcasted_iota(jnp.int32, (T, T), 0)
            == lax.broadcasted_iota(jnp.int32, (T, T), 1)
        ).astype(jnp.float32)
        att_t = lax.dot_general(att, eye, (((0,), (0,)), ((), ())),
                                preferred_element_type=jnp.float32)
        acc = att_t[:, 0:1] * nf3_ref[0]
        for t in range(1, T):
            acc = acc + att_t[:, t:t + 1] * nf3_ref[t]
        e2_ref[...] = (
            jnp.dot(acc, wi, preferred_element_type=jnp.float32) + bi
        )


def _attention(U, NF3, n_feature, Wu, bu, Wi, bi, att_W1, att_b1, att_W2, BB):
    B, D = U.shape
    T = NF3.shape[0]
    nblk = B // BB
    grid = nblk + 1

    def shifted(i):
        return jnp.maximum(i - 1, 0)

    body = functools.partial(_attn_body, T, BB)
    return pl.pallas_call(
        body,
        grid=(grid,),
        in_specs=[
            pl.BlockSpec((BB, D), lambda i: (shifted(i), 0)),
            pl.BlockSpec((T, BB, D), lambda i: (0, shifted(i), 0)),
            pl.BlockSpec((BB, D), lambda i: (shifted(i), 0)),
            pl.BlockSpec((D, D), lambda i: (0, 0)),
            pl.BlockSpec((1, D), lambda i: (0, 0)),
            pl.BlockSpec((D, D), lambda i: (0, 0)),
            pl.BlockSpec((1, D), lambda i: (0, 0)),
            pl.BlockSpec((D, D), lambda i: (0, 0)),
            pl.BlockSpec((D, D), lambda i: (0, 0)),
            pl.BlockSpec((1, D), lambda i: (0, 0)),
            pl.BlockSpec((1, D), lambda i: (0, 0)),
        ],
        out_specs=[
            pl.BlockSpec((BB, D), lambda i: (shifted(i), 0)),
            pl.BlockSpec((BB, D), lambda i: (i, 0)),
        ],
        out_shape=[
            jax.ShapeDtypeStruct((B, D), jnp.float32),
            jax.ShapeDtypeStruct((B + BB, D), jnp.float32),
        ],
    )(
        U, NF3, n_feature, Wu, bu.reshape(1, D),
        Wi, bi.reshape(1, D),
        att_W1[:D], att_W1[D:], att_b1.reshape(1, D),
        att_W2.reshape(1, D),
    )


def _make_scatter_kernel(B, D, NU, BB, H, E2_ROWS):
    n_chunks = NU // 128
    tail = NU - n_chunks * 128
    max_lc = n_chunks // NUM_WORKERS + 1
    slab = max_lc * 128
    n_batches = B // LANES
    init_steps = slab // LANES
    mesh = plsc.VectorSubcoreMesh(core_axis_name="c", subcore_axis_name="s")

    @functools.partial(
        pl.kernel,
        out_type=jax.ShapeDtypeStruct((NU, D), jnp.float32),
        mesh=mesh,
        name="sc_scatter",
        scratch_types=[
            pltpu.VMEM((B,), jnp.int32),
            pltpu.VMEM((slab,), jnp.int32),
            pltpu.VMEM((128, D), jnp.float32),
            pltpu.VMEM((128, D), jnp.float32),
            pltpu.SemaphoreType.DMA,
            pltpu.SemaphoreType.DMA,
            pltpu.SemaphoreType.DMA,
            pltpu.SemaphoreType.DMA,
        ],
        compiler_params=pltpu.CompilerParams(needs_layout_passes=False),
    )
    def scatter_kernel(nodes_hbm, e2_hbm, out_hbm, nodes_v, slot_v,
                       buf, buf1, sem, gs1, ws0, ws1):
        wid = lax.axis_index("s") * 2 + lax.axis_index("c")
        pltpu.sync_copy(nodes_hbm, nodes_v)

        lane = lax.iota(jnp.int32, LANES)

        def init(i, carry):
            slot_v[pl.ds(i * LANES, LANES)] = (i * LANES + lane) & (BB - 1)
            return carry

        lax.fori_loop(0, init_steps, init, 0)

        def build(b, carry):
            nv = nodes_v[pl.ds(b * LANES, LANES)]
            mask = ((nv >> 7) & (NUM_WORKERS - 1)) == wid
            local = ((nv >> 12) << 7) | (nv & 127)
            jv = b * LANES + lane
            vals = jv + BB + jnp.where(jv >= H, BB, 0)
            plsc.store_scatter(slot_v, [local], vals, mask=mask)
            return carry

        lax.fori_loop(0, n_batches, build, 0)

        _gather_rows_pipelined(
            e2_hbm, slot_v, out_hbm, 0, max_lc - 1,
            buf, buf1, sem, gs1, ws0, ws1,
            out_pos=lambda lc: (lc * NUM_WORKERS + wid) * 128)

        lc = max_lc - 1
        gc = lc * NUM_WORKERS + wid

        @pl.when(gc < n_chunks)
        def _full():
            sl = slot_v.at[pl.ds(lc * 128, 128)]
            pltpu.async_copy(e2_hbm.at[sl], buf, sem).wait()
            pltpu.sync_copy(buf, out_hbm.at[pl.ds(gc * 128, 128)])

        @pl.when(gc == n_chunks)
        def _tail():
            sl = slot_v.at[pl.ds(lc * 128, tail)]
            bt = buf.at[pl.ds(0, tail)]
            pltpu.async_copy(e2_hbm.at[sl], bt, sem).wait()
            pltpu.sync_copy(bt, out_hbm.at[pl.ds(gc * 128, tail)])

    return scatter_kernel


def kernel(nodes, n_feature, neigh_idx, u_table, i_table,
           Wu, bu, Wi, bi, att_W1, att_b1, att_W2, att_b2):
    B, T = neigh_idx.shape
    NU, DF = u_table.shape
    NI = i_table.shape[0]
    D = Wu.shape[1]
    BB = 512

    H = B // 2
    gather = _make_gather_kernel(H, T, D, NI, NU)
    feas, e2s = [], []
    for h in range(2):
        sl = slice(h * H, (h + 1) * H)
        nidx_t = neigh_idx[sl].T.reshape(-1)
        NF, U = gather(i_table, u_table, nidx_t, nodes[sl])
        fea_h, e2_h = _attention(U, NF.reshape(T, H, D), n_feature[sl],
                                 Wu, bu, Wi, bi, att_W1, att_b1, att_W2, BB)
        feas.append(fea_h)
        e2s.append(e2_h)
    nodes_fea = jnp.concatenate(feas, axis=0)
    E2 = jnp.concatenate(e2s, axis=0)

    embed_matrix = _make_scatter_kernel(B, D, NU, BB, H, E2.shape[0])(
        nodes, E2)

    return (nodes_fea, embed_matrix)
